# Initial kernel scaffold; baseline (speedup 1.0000x reference)
import jax
import jax.numpy as jnp
from jax.experimental import pallas as pl


def kernel(sampled_edge_indices, temporal_features, W1, att1, W2, att2, W3, att3):
    raise NotImplementedError("write your pallas kernel here")


# speedup vs baseline: 7458.3082x; 7458.3082x over previous
import jax
import jax.numpy as jnp
from jax.experimental import pallas as pl


def _gat_kernel(adj_ref, x_ref, w1_ref, a1_ref, w2_ref, a2_ref, w3_ref,
                a3_ref, out_ref):
    B = adj_ref.shape[0]
    m_i = adj_ref[0]
    for b in range(1, B):
        m_i = m_i + adj_ref[b]
    m = m_i.astype(jnp.float32)
    mask = m > 0.0
    cnt = jnp.sum(m, axis=0, keepdims=True)
    inv_cnt = 1.0 / jnp.maximum(cnt, 1.0)

    x = x_ref[...]
    for w_ref, a_ref in ((w1_ref, a1_ref), (w2_ref, a2_ref), (w3_ref, a3_ref)):
        W = w_ref[...]
        att = a_ref[...]
        H = W.shape[1]
        h = jax.lax.dot_general(x, W, (((1,), (0,)), ((), ())),
                                preferred_element_type=jnp.float32)
        a_dst = jax.lax.dot_general(att[:H], h, (((0,), (1,)), ((), ())),
                                    preferred_element_type=jnp.float32)
        a_src = jax.lax.dot_general(h, att[H:], (((1,), (0,)), ((), ())),
                                    preferred_element_type=jnp.float32)
        A = a_src + a_dst
        A = jnp.where(A >= 0.0, A, 0.2 * A)
        Amax = jnp.max(jnp.where(mask, A, -jnp.inf), axis=0, keepdims=True)
        Amax = jnp.where(jnp.isfinite(Amax), Amax, 0.0)
        P = m * jnp.exp(jnp.where(mask, A - Amax, 0.0))
        denom = jnp.sum(P, axis=0, keepdims=True)
        s = jax.lax.dot_general(P, h, (((0,), (0,)), ((), ())),
                                preferred_element_type=jnp.float32)
        x = s * (1.0 / (denom + 1e-16) * inv_cnt).reshape(-1, 1)
        x = jnp.where(x > 0.0, x, jnp.exp(x) - 1.0)
    out_ref[...] = x


def kernel(sampled_edge_indices, temporal_features, W1, att1, W2, att2, W3, att3):
    B, N, D = temporal_features.shape
    O = W3.shape[1]
    x0 = temporal_features[0]
    out = pl.pallas_call(
        _gat_kernel,
        out_shape=jax.ShapeDtypeStruct((N, O), jnp.float32),
    )(sampled_edge_indices, x0, W1, att1, W2, att2, W3, att3)
    full = jnp.zeros((B, N, O), jnp.float32)
    return full.at[0].set(out)
